# baseline (speedup 1.0000x reference)
import functools

import jax
import jax.numpy as jnp
from jax import lax
from jax.experimental import pallas as pl
from jax.experimental.pallas import tpu as pltpu
from jax.experimental.pallas import tpu_sc as plsc

B, S, A, G, NB, D = 16, 1024, 64, 128, 2048, 256
F3 = 3 * G
NB_SC = 512
NB_TC = NB - NB_SC
WPB = 2
RSC = NB_SC // WPB
CH = 64


def _stage1_body(subs_ref, vf_ref, eoh_ref, mask_ref,
                 W_ihT_ref, W_hhT_ref, b_ih_ref, b_hh_ref,
                 ln_sub_g_ref, ln_sub_b_ref, ln_post_g_ref, ln_post_b_ref,
                 l1_WT_ref, l1_b_ref, l2_WT_ref, l2_b_ref,
                 ln_pre_g_ref, ln_pre_b_ref, score_W_ref, score_b_ref,
                 probs_ref):
    f32 = jnp.float32
    subs = subs_ref[0]
    mask = mask_ref[0]
    subs_m = subs * mask
    vf = vf_ref[0]

    sws = jnp.dot(subs_m, vf, preferred_element_type=f32)
    inv_size = 1.0 / (jnp.sum(subs_m, axis=1, keepdims=True) + 1e-4)
    mean = sws * inv_size

    m = jnp.mean(mean, axis=-1, keepdims=True)
    v = jnp.mean((mean - m) ** 2, axis=-1, keepdims=True)
    h = (mean - m) * lax.rsqrt(v + 1e-5) * ln_sub_g_ref[0] + ln_sub_b_ref[0]

    r8 = lax.broadcasted_iota(jnp.int32, (8, G), 0)
    c8 = lax.broadcasted_iota(jnp.int32, (8, G), 1)
    P8 = jnp.where((c8 // 20 == r8) & (c8 < 100), 1.0, 0.0).astype(f32)
    EP = jnp.dot(eoh_ref[0], P8, preferred_element_type=f32)
    T = jnp.dot(subs, EP, preferred_element_type=f32)
    col1 = lax.broadcasted_iota(jnp.int32, (1, G), 1)
    colmod = (col1 % 20).astype(f32)
    valid = col1 < 100
    x = jnp.where((colmod <= T) & valid, 1.0, 0.0)

    gi = jnp.dot(x, W_ihT_ref[...], preferred_element_type=f32) + b_ih_ref[0]
    gh = jnp.dot(h, W_hhT_ref[...], preferred_element_type=f32) + b_hh_ref[0]
    r = jax.nn.sigmoid(gi[:, :G] + gh[:, :G])
    z = jax.nn.sigmoid(gi[:, G:2 * G] + gh[:, G:2 * G])
    n = jnp.tanh(gi[:, 2 * G:] + r * gh[:, 2 * G:])
    comb = (1.0 - z) * n + z * h

    m2 = jnp.mean(comb, axis=-1, keepdims=True)
    v2 = jnp.mean((comb - m2) ** 2, axis=-1, keepdims=True)
    y = (comb - m2) * lax.rsqrt(v2 + 1e-5) * ln_post_g_ref[0] + ln_post_b_ref[0]

    y = jax.nn.relu(jnp.dot(y, l1_WT_ref[...], preferred_element_type=f32)
                    + l1_b_ref[0])
    y = jax.nn.relu(jnp.dot(y, l2_WT_ref[...], preferred_element_type=f32)
                    + l2_b_ref[0])
    m3 = jnp.mean(y, axis=-1, keepdims=True)
    v3 = jnp.mean((y - m3) ** 2, axis=-1, keepdims=True)
    y = (y - m3) * lax.rsqrt(v3 + 1e-5) * ln_pre_g_ref[0] + ln_pre_b_ref[0]

    scores = (lax.dot_general(score_W_ref[...], y, (((1,), (1,)), ((), ())),
                              preferred_element_type=f32)
              + score_b_ref[...])

    e = jnp.exp(scores - jnp.max(scores, axis=1, keepdims=True))
    probs_ref[pl.ds(pl.program_id(0), 1), :] = e / jnp.sum(e, axis=1,
                                                           keepdims=True)


def _stage2_body(mm_ref, probs_ref, out_ref):
    b = pl.program_id(0)
    out_ref[pl.ds(b, 1), :] = lax.dot_general(
        probs_ref[pl.ds(b, 1), :], mm_ref[0], (((1,), (1,)), ((), ())),
        preferred_element_type=jnp.float32)


def _sc_matvec_body(mm_hbm, probs_hbm, out_hbm, probs_v, chunk_v, res_v):
    f32 = jnp.float32
    wid = lax.axis_index("s") * 2 + lax.axis_index("c")
    b = wid // WPB
    row0 = NB_TC + (wid % WPB) * RSC
    pltpu.sync_copy(probs_hbm.at[b], probs_v)

    def chunk_loop(ci, _):
        pltpu.sync_copy(mm_hbm.at[b, pl.ds(row0 + ci * CH, CH)], chunk_v)
        for rr in range(CH):

            def in_loop(j8, acc):
                base = j8 * 128
                for jj in range(8):
                    o = base + jj * 16
                    acc = acc + (chunk_v[rr, pl.ds(o, 16)]
                                 * probs_v[pl.ds(o, 16)])
                return acc

            acc = lax.fori_loop(0, S // 128, in_loop, jnp.zeros((16,), f32))
            res_v[pl.ds((ci * CH + rr) * 16, 16)] = acc
        return 0

    lax.fori_loop(0, RSC // CH, chunk_loop, 0)
    pltpu.sync_copy(res_v,
                    out_hbm.at[b, pl.ds((wid % WPB) * RSC * 16, RSC * 16)])


def _sc_matvec(mm, probs):
    mesh = plsc.VectorSubcoreMesh(core_axis_name="c", subcore_axis_name="s")
    kern = functools.partial(
        pl.kernel, mesh=mesh,
        out_type=jax.ShapeDtypeStruct((B, NB_SC * 16), jnp.float32),
        scratch_types=[
            pltpu.VMEM((S,), jnp.float32),
            pltpu.VMEM((CH, S), jnp.float32),
            pltpu.VMEM((RSC * 16,), jnp.float32),
        ],
    )(_sc_matvec_body)
    return kern(mm, probs)


def kernel(vert_feat_in, vert_mask_in, vert_element_oh, adj_oh, atom_subsets,
           atom_subsets_peaks, sparse_mass_matrix, W_ih, W_hh, b_ih, b_hh,
           ln_sub_g, ln_sub_b, ln_post_g, ln_post_b, l1_W, l1_b, l2_W, l2_b,
           ln_pre_g, ln_pre_b, score_W, score_b):
    f32 = jnp.float32
    mask3 = vert_mask_in.reshape(B, 1, A)
    eoh8 = jnp.pad(vert_element_oh, ((0, 0), (0, 0), (0, 3)))
    W_ihT = jnp.pad(W_ih, ((0, 0), (0, G - 100))).T
    W_hhT = W_hh.T
    l1_WT = l1_W.T
    l2_WT = l2_W.T
    row = lambda a: a.reshape(1, -1)

    full = lambda shp: pl.BlockSpec(shp, lambda b: (0,) * len(shp))
    probs = pl.pallas_call(
        _stage1_body,
        grid=(B,),
        in_specs=[
            pl.BlockSpec((1, S, A), lambda b: (b, 0, 0)),
            pl.BlockSpec((1, A, G), lambda b: (b, 0, 0)),
            pl.BlockSpec((1, A, 8), lambda b: (b, 0, 0)),
            pl.BlockSpec((1, 1, A), lambda b: (b, 0, 0)),
            full((G, F3)), full((G, F3)), full((1, F3)), full((1, F3)),
            full((1, G)), full((1, G)), full((1, G)), full((1, G)),
            full((G, D)), full((1, D)), full((D, D)), full((1, D)),
            full((1, D)), full((1, D)), full((1, D)), full((1, 1)),
        ],
        out_specs=pl.BlockSpec((B, S), lambda b: (0, 0)),
        out_shape=jax.ShapeDtypeStruct((B, S), f32),
    )(atom_subsets, vert_feat_in, eoh8, mask3,
      W_ihT, W_hhT, row(b_ih), row(b_hh),
      row(ln_sub_g), row(ln_sub_b), row(ln_post_g), row(ln_post_b),
      l1_WT, row(l1_b), l2_WT, row(l2_b),
      row(ln_pre_g), row(ln_pre_b), score_W, row(score_b))

    sc_partial = _sc_matvec(sparse_mass_matrix, probs)
    spect_sc = jnp.sum(sc_partial.reshape(B, NB_SC, 16), axis=-1)

    spect_tc = pl.pallas_call(
        _stage2_body,
        grid=(B, 1),
        in_specs=[
            pl.BlockSpec((1, NB_TC, S), lambda b, n: (b, n, 0)),
            pl.BlockSpec((B, S), lambda b, n: (0, 0)),
        ],
        out_specs=pl.BlockSpec((B, NB_TC), lambda b, n: (0, 0)),
        out_shape=jax.ShapeDtypeStruct((B, NB_TC), f32),
    )(sparse_mass_matrix, probs)

    spect = jnp.concatenate([spect_tc, spect_sc], axis=1)
    return (spect, probs)


# speedup vs baseline: 1.3242x; 1.3242x over previous
import jax
import jax.numpy as jnp
from jax import lax
from jax.experimental import pallas as pl

B, S, A, G, NB, D = 16, 1024, 64, 128, 2048, 256
F3 = 3 * G


def _stage1_body(subs_ref, vf_ref, eoh_ref, mask_ref,
                 W_ihT_ref, W_hhT_ref, b_ih_ref, b_hh_ref,
                 ln_sub_g_ref, ln_sub_b_ref, ln_post_g_ref, ln_post_b_ref,
                 l1_WT_ref, l1_b_ref, l2_WT_ref, l2_b_ref,
                 ln_pre_g_ref, ln_pre_b_ref, score_W_ref, score_b_ref,
                 probs_ref):
    f32 = jnp.float32
    subs = subs_ref[0]
    mask = mask_ref[0]
    subs_m = subs * mask
    vf = vf_ref[0]

    sws = jnp.dot(subs_m, vf, preferred_element_type=f32)
    inv_size = 1.0 / (jnp.sum(subs_m, axis=1, keepdims=True) + 1e-4)
    mean = sws * inv_size

    m = jnp.mean(mean, axis=-1, keepdims=True)
    v = jnp.mean((mean - m) ** 2, axis=-1, keepdims=True)
    h = (mean - m) * lax.rsqrt(v + 1e-5) * ln_sub_g_ref[0] + ln_sub_b_ref[0]

    r8 = lax.broadcasted_iota(jnp.int32, (8, G), 0)
    c8 = lax.broadcasted_iota(jnp.int32, (8, G), 1)
    P8 = jnp.where((c8 // 20 == r8) & (c8 < 100), 1.0, 0.0).astype(f32)
    EP = jnp.dot(eoh_ref[0], P8, preferred_element_type=f32)
    T = jnp.dot(subs, EP, preferred_element_type=f32)
    col1 = lax.broadcasted_iota(jnp.int32, (1, G), 1)
    colmod = (col1 % 20).astype(f32)
    valid = col1 < 100
    x = jnp.where((colmod <= T) & valid, 1.0, 0.0)

    gi = jnp.dot(x, W_ihT_ref[...], preferred_element_type=f32) + b_ih_ref[0]
    gh = jnp.dot(h, W_hhT_ref[...], preferred_element_type=f32) + b_hh_ref[0]
    r = jax.nn.sigmoid(gi[:, :G] + gh[:, :G])
    z = jax.nn.sigmoid(gi[:, G:2 * G] + gh[:, G:2 * G])
    n = jnp.tanh(gi[:, 2 * G:] + r * gh[:, 2 * G:])
    comb = (1.0 - z) * n + z * h

    m2 = jnp.mean(comb, axis=-1, keepdims=True)
    v2 = jnp.mean((comb - m2) ** 2, axis=-1, keepdims=True)
    y = (comb - m2) * lax.rsqrt(v2 + 1e-5) * ln_post_g_ref[0] + ln_post_b_ref[0]

    y = jax.nn.relu(jnp.dot(y, l1_WT_ref[...], preferred_element_type=f32)
                    + l1_b_ref[0])
    y = jax.nn.relu(jnp.dot(y, l2_WT_ref[...], preferred_element_type=f32)
                    + l2_b_ref[0])
    m3 = jnp.mean(y, axis=-1, keepdims=True)
    v3 = jnp.mean((y - m3) ** 2, axis=-1, keepdims=True)
    y = (y - m3) * lax.rsqrt(v3 + 1e-5) * ln_pre_g_ref[0] + ln_pre_b_ref[0]

    scores = (lax.dot_general(score_W_ref[...], y, (((1,), (1,)), ((), ())),
                              preferred_element_type=f32)
              + score_b_ref[...])

    e = jnp.exp(scores - jnp.max(scores, axis=1, keepdims=True))
    probs_ref[pl.ds(pl.program_id(0), 1), :] = e / jnp.sum(e, axis=1,
                                                           keepdims=True)


def _stage2_body(mm_ref, probs_ref, out_ref):
    b = pl.program_id(0)
    out_ref[pl.ds(b, 1), :] = lax.dot_general(
        probs_ref[pl.ds(b, 1), :], mm_ref[0], (((1,), (1,)), ((), ())),
        preferred_element_type=jnp.float32)


def kernel(vert_feat_in, vert_mask_in, vert_element_oh, adj_oh, atom_subsets,
           atom_subsets_peaks, sparse_mass_matrix, W_ih, W_hh, b_ih, b_hh,
           ln_sub_g, ln_sub_b, ln_post_g, ln_post_b, l1_W, l1_b, l2_W, l2_b,
           ln_pre_g, ln_pre_b, score_W, score_b):
    f32 = jnp.float32
    mask3 = vert_mask_in.reshape(B, 1, A)
    eoh8 = jnp.pad(vert_element_oh, ((0, 0), (0, 0), (0, 3)))
    W_ihT = jnp.pad(W_ih, ((0, 0), (0, G - 100))).T
    W_hhT = W_hh.T
    l1_WT = l1_W.T
    l2_WT = l2_W.T
    row = lambda a: a.reshape(1, -1)

    full = lambda shp: pl.BlockSpec(shp, lambda b: (0,) * len(shp))
    probs = pl.pallas_call(
        _stage1_body,
        grid=(B,),
        in_specs=[
            pl.BlockSpec((1, S, A), lambda b: (b, 0, 0)),
            pl.BlockSpec((1, A, G), lambda b: (b, 0, 0)),
            pl.BlockSpec((1, A, 8), lambda b: (b, 0, 0)),
            pl.BlockSpec((1, 1, A), lambda b: (b, 0, 0)),
            full((G, F3)), full((G, F3)), full((1, F3)), full((1, F3)),
            full((1, G)), full((1, G)), full((1, G)), full((1, G)),
            full((G, D)), full((1, D)), full((D, D)), full((1, D)),
            full((1, D)), full((1, D)), full((1, D)), full((1, 1)),
        ],
        out_specs=pl.BlockSpec((B, S), lambda b: (0, 0)),
        out_shape=jax.ShapeDtypeStruct((B, S), f32),
    )(atom_subsets, vert_feat_in, eoh8, mask3,
      W_ihT, W_hhT, row(b_ih), row(b_hh),
      row(ln_sub_g), row(ln_sub_b), row(ln_post_g), row(ln_post_b),
      l1_WT, row(l1_b), l2_WT, row(l2_b),
      row(ln_pre_g), row(ln_pre_b), score_W, row(score_b))

    BN = 2048
    spect = pl.pallas_call(
        _stage2_body,
        grid=(B, NB // BN),
        in_specs=[
            pl.BlockSpec((1, BN, S), lambda b, n: (b, n, 0)),
            pl.BlockSpec((B, S), lambda b, n: (0, 0)),
        ],
        out_specs=pl.BlockSpec((B, NB), lambda b, n: (0, 0)),
        out_shape=jax.ShapeDtypeStruct((B, NB), f32),
    )(sparse_mass_matrix, probs)

    return (spect, probs)


# speedup vs baseline: 1.4246x; 1.0759x over previous
import jax
import jax.numpy as jnp
from jax import lax
from jax.experimental import pallas as pl

B, S, A, G, NB, D = 16, 1024, 64, 128, 2048, 256
F3 = 3 * G


def _stage1_body(subs_ref, vf_ref, eoh_ref, mask_ref,
                 W_ih_ref, W_hh_ref, b_ih_ref, b_hh_ref,
                 ln_sub_g_ref, ln_sub_b_ref, ln_post_g_ref, ln_post_b_ref,
                 l1_W_ref, l1_b_ref, l2_W_ref, l2_b_ref,
                 ln_pre_g_ref, ln_pre_b_ref, score_W_ref, score_b_ref,
                 probs_ref):
    f32 = jnp.float32
    subs = subs_ref[0]
    mask = mask_ref[0]
    subs_m = subs * mask
    vf = vf_ref[0]

    sws = jnp.dot(subs_m, vf, preferred_element_type=f32)
    inv_size = 1.0 / (jnp.sum(subs_m, axis=1, keepdims=True) + 1e-4)
    mean = sws * inv_size

    m = jnp.mean(mean, axis=-1, keepdims=True)
    v = jnp.mean((mean - m) ** 2, axis=-1, keepdims=True)
    h = (mean - m) * lax.rsqrt(v + 1e-5) * ln_sub_g_ref[0] + ln_sub_b_ref[0]

    r8 = lax.broadcasted_iota(jnp.int32, (8, G), 0)
    c8 = lax.broadcasted_iota(jnp.int32, (8, G), 1)
    P8 = jnp.where((c8 // 20 == r8) & (c8 < 100), 1.0, 0.0).astype(f32)
    EP = jnp.dot(eoh_ref[0], P8, preferred_element_type=f32)
    T = jnp.dot(subs, EP, preferred_element_type=f32)
    col1 = lax.broadcasted_iota(jnp.int32, (1, G), 1)
    colmod = (col1 % 20).astype(f32)
    valid = col1 < 100
    x = jnp.where((colmod <= T) & valid, 1.0, 0.0)

    cR = (((1,), (1,)), ((), ()))
    gi = lax.dot_general(x, W_ih_ref[...], cR,
                         preferred_element_type=f32) + b_ih_ref[0]
    gh = lax.dot_general(h, W_hh_ref[...], cR,
                         preferred_element_type=f32) + b_hh_ref[0]
    r = jax.nn.sigmoid(gi[:, :G] + gh[:, :G])
    z = jax.nn.sigmoid(gi[:, G:2 * G] + gh[:, G:2 * G])
    n = jnp.tanh(gi[:, 2 * G:] + r * gh[:, 2 * G:])
    comb = (1.0 - z) * n + z * h

    m2 = jnp.mean(comb, axis=-1, keepdims=True)
    v2 = jnp.mean((comb - m2) ** 2, axis=-1, keepdims=True)
    y = (comb - m2) * lax.rsqrt(v2 + 1e-5) * ln_post_g_ref[0] + ln_post_b_ref[0]

    y = jax.nn.relu(lax.dot_general(y, l1_W_ref[...], cR,
                                    preferred_element_type=f32)
                    + l1_b_ref[0])
    y = jax.nn.relu(lax.dot_general(y, l2_W_ref[...], cR,
                                    preferred_element_type=f32)
                    + l2_b_ref[0])
    m3 = jnp.mean(y, axis=-1, keepdims=True)
    v3 = jnp.mean((y - m3) ** 2, axis=-1, keepdims=True)
    y = (y - m3) * lax.rsqrt(v3 + 1e-5) * ln_pre_g_ref[0] + ln_pre_b_ref[0]

    scores = (lax.dot_general(score_W_ref[...], y, (((1,), (1,)), ((), ())),
                              preferred_element_type=f32)
              + score_b_ref[...])

    e = jnp.exp(scores - jnp.max(scores, axis=1, keepdims=True))
    probs_ref[pl.ds(pl.program_id(0), 1), :] = e / jnp.sum(e, axis=1,
                                                           keepdims=True)


def _stage2_body(mm_ref, probs_ref, out_ref):
    b = pl.program_id(0)
    out_ref[pl.ds(b, 1), :] = lax.dot_general(
        probs_ref[pl.ds(b, 1), :], mm_ref[0], (((1,), (1,)), ((), ())),
        preferred_element_type=jnp.float32)


def kernel(vert_feat_in, vert_mask_in, vert_element_oh, adj_oh, atom_subsets,
           atom_subsets_peaks, sparse_mass_matrix, W_ih, W_hh, b_ih, b_hh,
           ln_sub_g, ln_sub_b, ln_post_g, ln_post_b, l1_W, l1_b, l2_W, l2_b,
           ln_pre_g, ln_pre_b, score_W, score_b):
    f32 = jnp.float32
    mask3 = vert_mask_in.reshape(B, 1, A)
    eoh8 = jnp.pad(vert_element_oh, ((0, 0), (0, 0), (0, 3)))
    W_ihp = jnp.pad(W_ih, ((0, 0), (0, G - 100)))
    row = lambda a: a.reshape(1, -1)

    full = lambda shp: pl.BlockSpec(shp, lambda b: (0,) * len(shp))
    probs = pl.pallas_call(
        _stage1_body,
        grid=(B,),
        in_specs=[
            pl.BlockSpec((1, S, A), lambda b: (b, 0, 0)),
            pl.BlockSpec((1, A, G), lambda b: (b, 0, 0)),
            pl.BlockSpec((1, A, 8), lambda b: (b, 0, 0)),
            pl.BlockSpec((1, 1, A), lambda b: (b, 0, 0)),
            full((F3, G)), full((F3, G)), full((1, F3)), full((1, F3)),
            full((1, G)), full((1, G)), full((1, G)), full((1, G)),
            full((D, G)), full((1, D)), full((D, D)), full((1, D)),
            full((1, D)), full((1, D)), full((1, D)), full((1, 1)),
        ],
        out_specs=pl.BlockSpec((B, S), lambda b: (0, 0)),
        out_shape=jax.ShapeDtypeStruct((B, S), f32),
    )(atom_subsets, vert_feat_in, eoh8, mask3,
      W_ihp, W_hh, row(b_ih), row(b_hh),
      row(ln_sub_g), row(ln_sub_b), row(ln_post_g), row(ln_post_b),
      l1_W, row(l1_b), l2_W, row(l2_b),
      row(ln_pre_g), row(ln_pre_b), score_W, row(score_b))

    BN = 2048
    spect = pl.pallas_call(
        _stage2_body,
        grid=(B, NB // BN),
        in_specs=[
            pl.BlockSpec((1, BN, S), lambda b, n: (b, n, 0)),
            pl.BlockSpec((B, S), lambda b, n: (0, 0)),
        ],
        out_specs=pl.BlockSpec((B, NB), lambda b, n: (0, 0)),
        out_shape=jax.ShapeDtypeStruct((B, NB), f32),
    )(sparse_mass_matrix, probs)

    return (spect, probs)
